# baseline (speedup 1.0000x reference)
import functools

import jax
import jax.numpy as jnp
from jax import lax
from jax.experimental import pallas as pl
from jax.experimental.pallas import tpu as pltpu
from jax.experimental.pallas import tpu_sc as plsc

K_DIM = 8192
E_DIM = 256
BETA = 0.25
N_ELEM = 8 * 32 * 32 * 256

TM = 512
TN = 512


def _argmin_body(zn_ref, wn_ref, z_ref, w_ref, idx_ref, run_min, run_idx):
    j = pl.program_id(1)
    nj = pl.num_programs(1)

    @pl.when(j == 0)
    def _init():
        run_min[:] = jnp.full((TM, 4), jnp.inf, jnp.float32)
        run_idx[:] = jnp.zeros((TM, 4), jnp.int32)

    m = lax.dot_general(z_ref[:], w_ref[:],
                        (((1,), (1,)), ((), ())),
                        preferred_element_type=jnp.float32)
    d = (zn_ref[:] + wn_ref[:]) - 2.0 * m
    bmin = jnp.min(d, axis=1, keepdims=True)
    col = j * TN + lax.broadcasted_iota(jnp.int32, (TM, TN), 1)
    cand = jnp.where(d == bmin, col, K_DIM)
    barg = jnp.min(cand, axis=1, keepdims=True)
    new_mins = []
    new_idxs = []
    for p in range(4):
        run_v = run_min[:, p:p + 1]
        run_i = run_idx[:, p:p + 1]
        upd = (j // 4 == p) & (bmin < run_v)
        new_mins.append(jnp.where(upd, bmin, run_v))
        new_idxs.append(jnp.where(upd, barg, run_i))
    run_min[:] = jnp.concatenate(new_mins, axis=1)
    run_idx[:] = jnp.concatenate(new_idxs, axis=1)

    @pl.when(j == nj - 1)
    def _out():
        acc_v = run_min[:, 0:1]
        acc_a = run_idx[:, 0:1]
        for p in range(1, 4):
            v = run_min[:, p:p + 1]
            a = run_idx[:, p:p + 1]
            q = acc_v.astype(jnp.bfloat16).astype(jnp.float32)
            upd = v < q
            acc_v = jnp.where(upd, v, acc_v)
            acc_a = jnp.where(upd, a, acc_a)
        idx_ref[:] = acc_a


def _vq_argmin(z_bf, w_bf, zn, wn):
    grid = (K_DIM // TM, K_DIM // TN)
    return pl.pallas_call(
        _argmin_body,
        grid=grid,
        in_specs=[
            pl.BlockSpec((TM, 1), lambda i, j: (i, 0)),
            pl.BlockSpec((1, TN), lambda i, j: (0, j)),
            pl.BlockSpec((TM, E_DIM), lambda i, j: (i, 0)),
            pl.BlockSpec((TN, E_DIM), lambda i, j: (j, 0)),
        ],
        out_specs=pl.BlockSpec((TM, 1), lambda i, j: (i, 0)),
        out_shape=jax.ShapeDtypeStruct((K_DIM, 1), jnp.int32),
        scratch_shapes=[
            pltpu.VMEM((TM, 4), jnp.float32),
            pltpu.VMEM((TM, 4), jnp.int32),
        ],
        compiler_params=pltpu.CompilerParams(
            dimension_semantics=("parallel", "arbitrary"),
        ),
    )(zn, wn, z_bf, w_bf)


def _make_sc_gather():
    info = plsc.get_sparse_core_info()
    NC, NS = info.num_cores, info.num_subcores
    NW = NC * NS
    b_per_w = K_DIM // NW
    chunk = 128
    n_chunks = b_per_w // chunk
    mesh = plsc.VectorSubcoreMesh(core_axis_name="c", subcore_axis_name="s")

    @functools.partial(
        pl.kernel, mesh=mesh,
        out_type=jax.ShapeDtypeStruct((K_DIM, E_DIM), jnp.float32),
        scratch_types=[
            pltpu.VMEM((chunk,), jnp.int32),
            pltpu.VMEM((chunk, E_DIM), jnp.float32),
            pltpu.SemaphoreType.DMA,
        ],
    )
    def gather(table_hbm, idx_hbm, out_hbm, idx_v, rows_v, sem):
        wid = lax.axis_index("s") * NC + lax.axis_index("c")
        for c in range(n_chunks):
            base = wid * b_per_w + c * chunk
            pltpu.sync_copy(idx_hbm.at[pl.ds(base, chunk)], idx_v)
            pltpu.async_copy(table_hbm.at[idx_v], rows_v, sem).wait()
            pltpu.sync_copy(rows_v, out_hbm.at[pl.ds(base, chunk)])

    return gather


_sc_gather = _make_sc_gather()


def _st_body(zp_ref, vq_ref, out_ref, loss_ref, acc):
    i = pl.program_id(0)

    @pl.when(i == 0)
    def _init():
        acc[0, 0] = 0.0

    zp = zp_ref[:]
    diff = vq_ref[:] - zp
    out_ref[:] = zp + diff
    acc[0, 0] += jnp.sum(diff * diff)

    @pl.when(i == pl.num_programs(0) - 1)
    def _out():
        loss_ref[:] = jnp.full((1, 1), acc[0, 0], jnp.float32)


def _st_loss(zp_flat, vq):
    grid = (K_DIM // TM,)
    return pl.pallas_call(
        _st_body,
        grid=grid,
        in_specs=[
            pl.BlockSpec((TM, E_DIM), lambda i: (i, 0)),
            pl.BlockSpec((TM, E_DIM), lambda i: (i, 0)),
        ],
        out_specs=[
            pl.BlockSpec((TM, E_DIM), lambda i: (i, 0)),
            pl.BlockSpec((1, 1), lambda i: (0, 0)),
        ],
        out_shape=[
            jax.ShapeDtypeStruct((K_DIM, E_DIM), jnp.float32),
            jax.ShapeDtypeStruct((1, 1), jnp.float32),
        ],
        scratch_shapes=[pltpu.SMEM((1, 1), jnp.float32)],
        compiler_params=pltpu.CompilerParams(
            dimension_semantics=("arbitrary",),
        ),
    )(zp_flat, vq)


def kernel(z, W):
    zp = jnp.transpose(z, (0, 2, 3, 1))
    z_flat = zp.reshape(-1, E_DIM)
    zn = jnp.sum(z_flat ** 2, axis=1, keepdims=True)
    wn = jnp.sum(W ** 2, axis=1).reshape(1, K_DIM)
    z_bf = z_flat.astype(jnp.bfloat16)
    w_bf = W.astype(jnp.bfloat16)

    idx = _vq_argmin(z_bf, w_bf, zn, wn).reshape(-1)

    Wb = W.astype(jnp.bfloat16).astype(jnp.float32)
    vq = _sc_gather(Wb, idx)

    vq_out_flat, s = _st_loss(z_flat, vq)
    mean_sq = s[0, 0] / N_ELEM
    loss = mean_sq + BETA * mean_sq
    vq_out = vq_out_flat.reshape(zp.shape)
    vq_out = jnp.transpose(vq_out, (0, 3, 1, 2))
    return (loss, vq_out)
